# baseline (speedup 1.0000x reference)
import functools

import jax
import jax.numpy as jnp
from jax import lax
from jax.experimental import pallas as pl
from jax.experimental.pallas import tpu as pltpu
from jax.experimental.pallas import tpu_sc as plsc

D_MODEL = 768
D_SAE = 24576
K = 128
N_TOK = 8192

BR = 1024
BC = 512

_PREC = lax.Precision.DEFAULT


def _encode_body(x_ref, w_ref, benc_ref, bdec_ref, out_ref, bias_scr):
    i = pl.program_id(0)
    j = pl.program_id(1)

    @pl.when(i == 0)
    def _():
        bias_scr[0, pl.ds(j * BC, BC)] = (
            benc_ref[...]
            - lax.dot_general(bdec_ref[...], w_ref[...],
                              (((1,), (1,)), ((), ())), precision=_PREC)
        )[0]

    bias = bias_scr[0, pl.ds(j * BC, BC)]
    acts = lax.dot_general(x_ref[...], w_ref[...],
                           (((1,), (1,)), ((), ())), precision=_PREC)
    out_ref[...] = jnp.maximum(acts + bias[None, :], 0.0)


def _encode(x, W_enc, b_enc, b_dec):
    grid = (x.shape[0] // BR, D_SAE // BC)
    return pl.pallas_call(
        _encode_body,
        grid=grid,
        in_specs=[
            pl.BlockSpec((BR, D_MODEL), lambda i, j: (i, 0)),
            pl.BlockSpec((BC, D_MODEL), lambda i, j: (j, 0)),
            pl.BlockSpec((1, BC), lambda i, j: (0, j)),
            pl.BlockSpec((1, D_MODEL), lambda i, j: (0, 0)),
        ],
        out_specs=pl.BlockSpec((BR, BC), lambda i, j: (i, j)),
        out_shape=jax.ShapeDtypeStruct((x.shape[0], D_SAE), jnp.float32),
        scratch_shapes=[pltpu.VMEM((1, D_SAE), jnp.float32)],
    )(x, W_enc, b_enc.reshape(1, D_SAE), b_dec.reshape(1, D_MODEL))


def _decode_body(enc_ref, w_ref, bdec_ref, out_ref):
    k = pl.program_id(1)
    acts = lax.dot_general(enc_ref[...], w_ref[...],
                           (((1,), (1,)), ((), ())), precision=_PREC)

    @pl.when(k == 0)
    def _():
        out_ref[...] = acts + bdec_ref[...]

    @pl.when(k != 0)
    def _():
        out_ref[...] += acts


def _decode(encoded, W_dec, b_dec):
    grid = (encoded.shape[0] // BR, D_SAE // BC)
    return pl.pallas_call(
        _decode_body,
        grid=grid,
        in_specs=[
            pl.BlockSpec((BR, BC), lambda i, k: (i, k)),
            pl.BlockSpec((D_MODEL, BC), lambda i, k: (0, k)),
            pl.BlockSpec((1, D_MODEL), lambda i, k: (0, 0)),
        ],
        out_specs=pl.BlockSpec((BR, D_MODEL), lambda i, k: (i, 0)),
        out_shape=jax.ShapeDtypeStruct((encoded.shape[0], D_MODEL), jnp.float32),
    )(encoded, W_dec, b_dec.reshape(1, D_MODEL))


NW = 32
NV = D_SAE // 16
U = 8
HB = 8192


def _scan_top(hist, start_block, target):
    iota16 = lax.iota(jnp.int32, 16)

    def sc_cond(c):
        i, cum = c
        return jnp.logical_and(cum < target, i >= 0)

    def sc_body(c):
        i, cum = c
        return i - 1, cum + jnp.sum(hist[pl.ds(i * 16, 16)])

    i_end, cum_end = lax.while_loop(
        sc_cond, sc_body, (start_block, jnp.int32(0)))
    found = cum_end >= target
    iv = jnp.maximum(i_end + 1, 0)
    h = hist[pl.ds(iv * 16, 16)]
    cum_above_blk = cum_end - jnp.sum(h)
    suffix = lax.rev(jnp.cumsum(lax.rev(h, (0,))), (0,)) + cum_above_blk
    m = suffix >= target
    lane = jnp.sum(m.astype(jnp.int32)) - 1
    bucket = iv * 16 + lane
    sfx_lane = cum_above_blk + jnp.sum(jnp.where(iota16 >= lane, h, 0))
    h_lane = jnp.sum(jnp.where(iota16 == lane, h, 0))
    above = sfx_lane - h_lane
    return found, bucket, above


def _zero_hist(hist, nbuckets):
    @plsc.parallel_loop(0, nbuckets // 16, unroll=U)
    def _(i):
        hist[pl.ds(i * 16, 16)] = jnp.zeros((16,), jnp.int32)


def _row_select(rows, outs, hist, roff, ooff):
    ones16 = jnp.ones((16,), jnp.int32)

    def bits_at(i):
        v = rows[pl.ds(roff + i * 16, 16)]
        return v, lax.bitcast_convert_type(v, jnp.int32)

    _zero_hist(hist, 8192)

    @plsc.parallel_loop(0, NV, unroll=U, carry=jnp.zeros((16,), jnp.int32))
    def mbv(i, mb):
        _, bits = bits_at(i)
        b = lax.shift_right_logical(bits, 18)
        plsc.addupdate_scatter(hist, [b], ones16, mask=bits >= 1)
        return jnp.maximum(mb, jnp.where(bits >= 1, b, 0))

    max_blk = lax.shift_right_logical(jnp.max(mbv), 4)
    found1, b1, above1 = _scan_top(hist, max_blk, jnp.int32(K))

    _zero_hist(hist, 512)
    need2 = jnp.int32(K) - above1

    @plsc.parallel_loop(0, NV, unroll=U)
    def _(i):
        _, bits = bits_at(i)
        m = lax.shift_right_logical(bits, 18) == b1
        b = jnp.bitwise_and(lax.shift_right_logical(bits, 9), jnp.int32(0x1FF))
        plsc.addupdate_scatter(hist, [b], ones16, mask=m)

    _f2, b2, above2 = _scan_top(hist, jnp.int32(31), need2)
    prefix22 = jnp.bitwise_or(lax.shift_left(b1, 9), b2)

    _zero_hist(hist, 512)
    need3 = need2 - above2

    @plsc.parallel_loop(0, NV, unroll=U)
    def _(i):
        _, bits = bits_at(i)
        m = lax.shift_right_logical(bits, 9) == prefix22
        b = jnp.bitwise_and(bits, jnp.int32(0x1FF))
        plsc.addupdate_scatter(hist, [b], ones16, mask=m)

    _f3, b3, _a3 = _scan_top(hist, jnp.int32(31), need3)

    t_cut = jnp.bitwise_or(lax.shift_left(prefix22, 9), b3)
    t_cut = jnp.where(found1, jnp.maximum(t_cut, 1), jnp.int32(1))

    @plsc.parallel_loop(0, NV, unroll=U)
    def _(i):
        v, bits = bits_at(i)
        m = bits >= t_cut
        outs[pl.ds(ooff + i * 16, 16)] = jnp.where(m, v, 0.0)


@functools.lru_cache(maxsize=None)
def _make_select(n_tok):
    ROWS_PER_W = n_tok // NW
    mesh = plsc.VectorSubcoreMesh(core_axis_name="c", subcore_axis_name="s",
                                  num_cores=2, num_subcores=16)

    @functools.partial(
        pl.kernel, mesh=mesh,
        out_type=jax.ShapeDtypeStruct((n_tok, D_SAE), jnp.float32),
        scratch_types=[
            pltpu.VMEM((2 * D_SAE,), jnp.float32),
            pltpu.VMEM((2 * D_SAE,), jnp.float32),
            pltpu.VMEM((HB,), jnp.int32),
            pltpu.SemaphoreType.DMA,
            pltpu.SemaphoreType.DMA,
            pltpu.SemaphoreType.DMA,
            pltpu.SemaphoreType.DMA,
        ],
        compiler_params=pltpu.CompilerParams(needs_layout_passes=False),
    )
    def select(pre_hbm, out_hbm, rows, outs, hist, si0, si1, so0, so1):
        wid = lax.axis_index("s") * 2 + lax.axis_index("c")
        base = wid * ROWS_PER_W
        isems = (si0, si1)
        osems = (so0, so1)

        pltpu.async_copy(pre_hbm.at[base], rows.at[pl.ds(0, D_SAE)], si0)

        def pair_body(r2, _):
            for b in range(2):
                r = 2 * r2 + b
                row = base + r
                roff = b * D_SAE
                pltpu.make_async_copy(
                    pre_hbm.at[row], rows.at[pl.ds(roff, D_SAE)],
                    isems[b]).wait()

                @pl.when(r + 1 < ROWS_PER_W)
                def _():
                    pltpu.async_copy(
                        pre_hbm.at[row + 1],
                        rows.at[pl.ds((1 - b) * D_SAE, D_SAE)], isems[1 - b])

                @pl.when(r >= 2)
                def _():
                    pltpu.make_async_copy(
                        outs.at[pl.ds(roff, D_SAE)], out_hbm.at[row - 2],
                        osems[b]).wait()

                _row_select(rows, outs, hist, roff, roff)
                pltpu.async_copy(outs.at[pl.ds(roff, D_SAE)],
                                 out_hbm.at[row], osems[b])
            return 0
        lax.fori_loop(0, ROWS_PER_W // 2, pair_body, 0)

        for b in range(2):
            row = base + ROWS_PER_W - 2 + b
            pltpu.make_async_copy(outs.at[pl.ds(b * D_SAE, D_SAE)],
                                  out_hbm.at[row], osems[b]).wait()

    return select


NCHUNK = 8


def kernel(x, W_enc, b_enc, W_dec, b_dec):
    cs = N_TOK // NCHUNK
    recs, encs = [], []
    for c in range(NCHUNK):
        pre_c = _encode(lax.slice_in_dim(x, c * cs, (c + 1) * cs), W_enc,
                        b_enc, b_dec)
        enc_c = _make_select(cs)(pre_c)
        recs.append(_decode(enc_c, W_dec, b_dec))
        encs.append(enc_c)
    return jnp.concatenate(recs, 0), jnp.concatenate(encs, 0)


# speedup vs baseline: 1.1981x; 1.1981x over previous
import functools

import jax
import jax.numpy as jnp
from jax import lax
from jax.experimental import pallas as pl
from jax.experimental.pallas import tpu as pltpu
from jax.experimental.pallas import tpu_sc as plsc

D_MODEL = 768
D_SAE = 24576
K = 128
N_TOK = 8192

BR = 1024
BC = 512

_PREC = lax.Precision.DEFAULT


def _encode_body(x_ref, w_ref, benc_ref, bdec_ref, out_ref, bias_scr):
    i = pl.program_id(0)
    j = pl.program_id(1)

    @pl.when(i == 0)
    def _():
        bias_scr[0, pl.ds(j * BC, BC)] = (
            benc_ref[...]
            - lax.dot_general(bdec_ref[...], w_ref[...],
                              (((1,), (1,)), ((), ())), precision=_PREC)
        )[0]

    bias = bias_scr[0, pl.ds(j * BC, BC)]
    acts = lax.dot_general(x_ref[...], w_ref[...],
                           (((1,), (1,)), ((), ())), precision=_PREC)
    out_ref[...] = jnp.maximum(acts + bias[None, :], 0.0)


def _encode(x, W_enc, b_enc, b_dec):
    grid = (x.shape[0] // BR, D_SAE // BC)
    return pl.pallas_call(
        _encode_body,
        grid=grid,
        in_specs=[
            pl.BlockSpec((BR, D_MODEL), lambda i, j: (i, 0)),
            pl.BlockSpec((BC, D_MODEL), lambda i, j: (j, 0)),
            pl.BlockSpec((1, BC), lambda i, j: (0, j)),
            pl.BlockSpec((1, D_MODEL), lambda i, j: (0, 0)),
        ],
        out_specs=pl.BlockSpec((BR, BC), lambda i, j: (i, j)),
        out_shape=jax.ShapeDtypeStruct((x.shape[0], D_SAE), jnp.float32),
        scratch_shapes=[pltpu.VMEM((1, D_SAE), jnp.float32)],
    )(x, W_enc, b_enc.reshape(1, D_SAE), b_dec.reshape(1, D_MODEL))


def _decode_body(*refs):
    if len(refs) == 6:
        enc_ref, w_ref, bdec_ref, _prev, out_ref, encfull_ref = refs
    else:
        enc_ref, w_ref, bdec_ref, out_ref, encfull_ref = refs
    k = pl.program_id(1)
    encfull_ref[...] = enc_ref[...]
    acts = lax.dot_general(enc_ref[...], w_ref[...],
                           (((1,), (1,)), ((), ())), precision=_PREC)

    @pl.when(k == 0)
    def _():
        out_ref[...] = acts + bdec_ref[...]

    @pl.when(k != 0)
    def _():
        out_ref[...] += acts


def _decode_assemble(encoded_c, W_dec, b_dec, enc_prev, chunk):
    cs = encoded_c.shape[0]
    grid = (cs // BR, D_SAE // BC)
    row0 = chunk * cs // BR
    args = [encoded_c, W_dec, b_dec.reshape(1, D_MODEL)]
    aliases = {}
    if enc_prev is not None:
        args.append(enc_prev)
        aliases = {3: 1}
    return pl.pallas_call(
        _decode_body,
        grid=grid,
        in_specs=[
            pl.BlockSpec((BR, BC), lambda i, k: (i, k)),
            pl.BlockSpec((D_MODEL, BC), lambda i, k: (0, k)),
            pl.BlockSpec((1, D_MODEL), lambda i, k: (0, 0)),
        ] + ([pl.BlockSpec(memory_space=pl.MemorySpace.ANY)] if enc_prev is not None
             else []),
        out_specs=[
            pl.BlockSpec((BR, D_MODEL), lambda i, k: (i, 0)),
            pl.BlockSpec((BR, BC), lambda i, k: (row0 + i, k)),
        ],
        out_shape=[
            jax.ShapeDtypeStruct((cs, D_MODEL), jnp.float32),
            jax.ShapeDtypeStruct((N_TOK, D_SAE), jnp.float32),
        ],
        input_output_aliases=aliases,
    )(*args)


NW = 32
NV = D_SAE // 16
U = 8
HB = 8192


def _scan_top(hist, start_block, target):
    iota16 = lax.iota(jnp.int32, 16)

    def sc_cond(c):
        i, cum = c
        return jnp.logical_and(cum < target, i >= 0)

    def sc_body(c):
        i, cum = c
        return i - 1, cum + jnp.sum(hist[pl.ds(i * 16, 16)])

    i_end, cum_end = lax.while_loop(
        sc_cond, sc_body, (start_block, jnp.int32(0)))
    found = cum_end >= target
    iv = jnp.maximum(i_end + 1, 0)
    h = hist[pl.ds(iv * 16, 16)]
    cum_above_blk = cum_end - jnp.sum(h)
    suffix = lax.rev(jnp.cumsum(lax.rev(h, (0,))), (0,)) + cum_above_blk
    m = suffix >= target
    lane = jnp.sum(m.astype(jnp.int32)) - 1
    bucket = iv * 16 + lane
    sfx_lane = cum_above_blk + jnp.sum(jnp.where(iota16 >= lane, h, 0))
    h_lane = jnp.sum(jnp.where(iota16 == lane, h, 0))
    above = sfx_lane - h_lane
    return found, bucket, above


def _zero_hist(hist, nbuckets):
    @plsc.parallel_loop(0, nbuckets // 16, unroll=U)
    def _(i):
        hist[pl.ds(i * 16, 16)] = jnp.zeros((16,), jnp.int32)


def _row_select(rows, outs, hist, roff, ooff):
    ones16 = jnp.ones((16,), jnp.int32)

    def bits_at(i):
        v = rows[pl.ds(roff + i * 16, 16)]
        return v, lax.bitcast_convert_type(v, jnp.int32)

    _zero_hist(hist, 8192)

    @plsc.parallel_loop(0, NV, unroll=U, carry=jnp.zeros((16,), jnp.int32))
    def mbv(i, mb):
        _, bits = bits_at(i)
        b = lax.shift_right_logical(bits, 18)
        plsc.addupdate_scatter(hist, [b], ones16, mask=bits >= 1)
        return jnp.maximum(mb, jnp.where(bits >= 1, b, 0))

    max_blk = lax.shift_right_logical(jnp.max(mbv), 4)
    found1, b1, above1 = _scan_top(hist, max_blk, jnp.int32(K))

    _zero_hist(hist, 512)
    need2 = jnp.int32(K) - above1

    @plsc.parallel_loop(0, NV, unroll=U)
    def _(i):
        _, bits = bits_at(i)
        m = lax.shift_right_logical(bits, 18) == b1
        b = jnp.bitwise_and(lax.shift_right_logical(bits, 9), jnp.int32(0x1FF))
        plsc.addupdate_scatter(hist, [b], ones16, mask=m)

    _f2, b2, above2 = _scan_top(hist, jnp.int32(31), need2)
    prefix22 = jnp.bitwise_or(lax.shift_left(b1, 9), b2)

    _zero_hist(hist, 512)
    need3 = need2 - above2

    @plsc.parallel_loop(0, NV, unroll=U)
    def _(i):
        _, bits = bits_at(i)
        m = lax.shift_right_logical(bits, 9) == prefix22
        b = jnp.bitwise_and(bits, jnp.int32(0x1FF))
        plsc.addupdate_scatter(hist, [b], ones16, mask=m)

    _f3, b3, _a3 = _scan_top(hist, jnp.int32(31), need3)

    t_cut = jnp.bitwise_or(lax.shift_left(prefix22, 9), b3)
    t_cut = jnp.where(found1, jnp.maximum(t_cut, 1), jnp.int32(1))

    @plsc.parallel_loop(0, NV, unroll=U)
    def _(i):
        v, bits = bits_at(i)
        m = bits >= t_cut
        outs[pl.ds(ooff + i * 16, 16)] = jnp.where(m, v, 0.0)


@functools.lru_cache(maxsize=None)
def _make_select(n_tok):
    ROWS_PER_W = n_tok // NW
    mesh = plsc.VectorSubcoreMesh(core_axis_name="c", subcore_axis_name="s",
                                  num_cores=2, num_subcores=16)

    @functools.partial(
        pl.kernel, mesh=mesh,
        out_type=jax.ShapeDtypeStruct((n_tok, D_SAE), jnp.float32),
        scratch_types=[
            pltpu.VMEM((2 * D_SAE,), jnp.float32),
            pltpu.VMEM((2 * D_SAE,), jnp.float32),
            pltpu.VMEM((HB,), jnp.int32),
            pltpu.SemaphoreType.DMA,
            pltpu.SemaphoreType.DMA,
            pltpu.SemaphoreType.DMA,
            pltpu.SemaphoreType.DMA,
        ],
        compiler_params=pltpu.CompilerParams(needs_layout_passes=False),
    )
    def select(pre_hbm, out_hbm, rows, outs, hist, si0, si1, so0, so1):
        wid = lax.axis_index("s") * 2 + lax.axis_index("c")
        base = wid * ROWS_PER_W
        isems = (si0, si1)
        osems = (so0, so1)

        pltpu.async_copy(pre_hbm.at[base], rows.at[pl.ds(0, D_SAE)], si0)

        def pair_body(r2, _):
            for b in range(2):
                r = 2 * r2 + b
                row = base + r
                roff = b * D_SAE
                pltpu.make_async_copy(
                    pre_hbm.at[row], rows.at[pl.ds(roff, D_SAE)],
                    isems[b]).wait()

                @pl.when(r + 1 < ROWS_PER_W)
                def _():
                    pltpu.async_copy(
                        pre_hbm.at[row + 1],
                        rows.at[pl.ds((1 - b) * D_SAE, D_SAE)], isems[1 - b])

                @pl.when(r >= 2)
                def _():
                    pltpu.make_async_copy(
                        outs.at[pl.ds(roff, D_SAE)], out_hbm.at[row - 2],
                        osems[b]).wait()

                _row_select(rows, outs, hist, roff, roff)
                pltpu.async_copy(outs.at[pl.ds(roff, D_SAE)],
                                 out_hbm.at[row], osems[b])
            return 0
        lax.fori_loop(0, ROWS_PER_W // 2, pair_body, 0)

        for b in range(2):
            row = base + ROWS_PER_W - 2 + b
            pltpu.make_async_copy(outs.at[pl.ds(b * D_SAE, D_SAE)],
                                  out_hbm.at[row], osems[b]).wait()

    return select


NCHUNK = 8


def kernel(x, W_enc, b_enc, W_dec, b_dec):
    cs = N_TOK // NCHUNK
    recs, enc_full = [], None
    for c in range(NCHUNK):
        pre_c = _encode(lax.slice_in_dim(x, c * cs, (c + 1) * cs), W_enc,
                        b_enc, b_dec)
        enc_c = _make_select(cs)(pre_c)
        rec_c, enc_full = _decode_assemble(enc_c, W_dec, b_dec, enc_full, c)
        recs.append(rec_c)
    return jnp.concatenate(recs, 0), enc_full


# speedup vs baseline: 1.4233x; 1.1879x over previous
import functools

import jax
import jax.numpy as jnp
from jax import lax
from jax.experimental import pallas as pl
from jax.experimental.pallas import tpu as pltpu
from jax.experimental.pallas import tpu_sc as plsc

D_MODEL = 768
D_SAE = 24576
K = 128
N_TOK = 8192

BR = 1024
BC = 512

_PREC = lax.Precision.DEFAULT


def _encode_body(x_ref, w_ref, benc_ref, bdec_ref, out_ref, bias_scr):
    i = pl.program_id(0)
    j = pl.program_id(1)

    @pl.when(i == 0)
    def _():
        bias_scr[0, pl.ds(j * BC, BC)] = (
            benc_ref[...]
            - lax.dot_general(bdec_ref[...], w_ref[...],
                              (((1,), (1,)), ((), ())), precision=_PREC)
        )[0]

    bias = bias_scr[0, pl.ds(j * BC, BC)]
    acts = lax.dot_general(x_ref[...], w_ref[...],
                           (((1,), (1,)), ((), ())), precision=_PREC)
    out_ref[...] = jnp.maximum(acts + bias[None, :], 0.0)


def _encode(x, W_enc, b_enc, b_dec):
    grid = (x.shape[0] // BR, D_SAE // BC)
    return pl.pallas_call(
        _encode_body,
        grid=grid,
        in_specs=[
            pl.BlockSpec((BR, D_MODEL), lambda i, j: (i, 0)),
            pl.BlockSpec((BC, D_MODEL), lambda i, j: (j, 0)),
            pl.BlockSpec((1, BC), lambda i, j: (0, j)),
            pl.BlockSpec((1, D_MODEL), lambda i, j: (0, 0)),
        ],
        out_specs=pl.BlockSpec((BR, BC), lambda i, j: (i, j)),
        out_shape=jax.ShapeDtypeStruct((x.shape[0], D_SAE), jnp.float32),
        scratch_shapes=[pltpu.VMEM((1, D_SAE), jnp.float32)],
    )(x, W_enc, b_enc.reshape(1, D_SAE), b_dec.reshape(1, D_MODEL))


def _decode_body(*refs):
    if len(refs) == 6:
        enc_ref, w_ref, bdec_ref, _prev, out_ref, encfull_ref = refs
    else:
        enc_ref, w_ref, bdec_ref, out_ref, encfull_ref = refs
    k = pl.program_id(1)
    encfull_ref[...] = enc_ref[...]
    acts = lax.dot_general(enc_ref[...], w_ref[...],
                           (((1,), (1,)), ((), ())), precision=_PREC)

    @pl.when(k == 0)
    def _():
        out_ref[...] = acts + bdec_ref[...]

    @pl.when(k != 0)
    def _():
        out_ref[...] += acts


def _decode_assemble(encoded_c, W_dec, b_dec, enc_prev, chunk):
    cs = encoded_c.shape[0]
    grid = (cs // BR, D_SAE // BC)
    row0 = chunk * cs // BR
    args = [encoded_c, W_dec, b_dec.reshape(1, D_MODEL)]
    aliases = {}
    if enc_prev is not None:
        args.append(enc_prev)
        aliases = {3: 1}
    return pl.pallas_call(
        _decode_body,
        grid=grid,
        in_specs=[
            pl.BlockSpec((BR, BC), lambda i, k: (i, k)),
            pl.BlockSpec((D_MODEL, BC), lambda i, k: (0, k)),
            pl.BlockSpec((1, D_MODEL), lambda i, k: (0, 0)),
        ] + ([pl.BlockSpec(memory_space=pl.MemorySpace.ANY)] if enc_prev is not None
             else []),
        out_specs=[
            pl.BlockSpec((BR, D_MODEL), lambda i, k: (i, 0)),
            pl.BlockSpec((BR, BC), lambda i, k: (row0 + i, k)),
        ],
        out_shape=[
            jax.ShapeDtypeStruct((cs, D_MODEL), jnp.float32),
            jax.ShapeDtypeStruct((N_TOK, D_SAE), jnp.float32),
        ],
        input_output_aliases=aliases,
    )(*args)


NW = 32
NV = D_SAE // 16
U = 8
HB = 8192


def _scan_top(hist, start_block, target):
    iota16 = lax.iota(jnp.int32, 16)

    def sc_cond(c):
        i, cum = c
        return jnp.logical_and(cum < target, i >= 0)

    def sc_body(c):
        i, cum = c
        return i - 1, cum + jnp.sum(hist[pl.ds(i * 16, 16)])

    i_end, cum_end = lax.while_loop(
        sc_cond, sc_body, (start_block, jnp.int32(0)))
    found = cum_end >= target
    iv = jnp.maximum(i_end + 1, 0)
    h = hist[pl.ds(iv * 16, 16)]
    cum_above_blk = cum_end - jnp.sum(h)
    suffix = lax.rev(jnp.cumsum(lax.rev(h, (0,))), (0,)) + cum_above_blk
    m = suffix >= target
    lane = jnp.sum(m.astype(jnp.int32)) - 1
    bucket = iv * 16 + lane
    sfx_lane = cum_above_blk + jnp.sum(jnp.where(iota16 >= lane, h, 0))
    h_lane = jnp.sum(jnp.where(iota16 == lane, h, 0))
    above = sfx_lane - h_lane
    return found, bucket, above, h_lane


def _zero_hist(hist, nbuckets):
    @plsc.parallel_loop(0, nbuckets // 16, unroll=U)
    def _(i):
        hist[pl.ds(i * 16, 16)] = jnp.zeros((16,), jnp.int32)


def _row_select(rows, outs, hist, roff, ooff):
    ones16 = jnp.ones((16,), jnp.int32)

    def bits_at(i):
        v = rows[pl.ds(roff + i * 16, 16)]
        return v, lax.bitcast_convert_type(v, jnp.int32)

    _zero_hist(hist, 8192)

    @plsc.parallel_loop(0, NV, unroll=U, carry=jnp.zeros((16,), jnp.int32))
    def mbv(i, mb):
        _, bits = bits_at(i)
        b = lax.shift_right_logical(bits, 18)
        plsc.addupdate_scatter(hist, [b], ones16, mask=bits >= 1)
        return jnp.maximum(mb, jnp.where(bits >= 1, b, 0))

    max_blk = lax.shift_right_logical(jnp.max(mbv), 4)
    found1, b1, above1, cnt1 = _scan_top(hist, max_blk, jnp.int32(K))
    need2 = jnp.int32(K) - above1

    def _refine_l2(_):
        _zero_hist(hist, 512)

        @plsc.parallel_loop(0, NV, unroll=U)
        def _(i):
            _, bits = bits_at(i)
            m = lax.shift_right_logical(bits, 18) == b1
            b = jnp.bitwise_and(lax.shift_right_logical(bits, 9),
                                jnp.int32(0x1FF))
            plsc.addupdate_scatter(hist, [b], ones16, mask=m)

        _f2, b2, above2, cnt2 = _scan_top(hist, jnp.int32(31), need2)
        prefix22 = jnp.bitwise_or(lax.shift_left(b1, 9), b2)
        need3 = need2 - above2

        def _refine_l3(_):
            _zero_hist(hist, 512)

            @plsc.parallel_loop(0, NV, unroll=U)
            def _(i):
                _, bits = bits_at(i)
                m = lax.shift_right_logical(bits, 9) == prefix22
                b = jnp.bitwise_and(bits, jnp.int32(0x1FF))
                plsc.addupdate_scatter(hist, [b], ones16, mask=m)

            _f3, b3, _a3, _c3 = _scan_top(hist, jnp.int32(31), need3)
            return jnp.bitwise_or(lax.shift_left(prefix22, 9), b3)

        return lax.cond(cnt2 == need3, lambda _: lax.shift_left(prefix22, 9),
                        _refine_l3, 0)

    skip_l2 = jnp.logical_or(jnp.logical_not(found1), cnt1 == need2)
    t_cut = lax.cond(skip_l2, lambda _: lax.shift_left(b1, 18), _refine_l2, 0)
    t_cut = jnp.where(found1, jnp.maximum(t_cut, 1), jnp.int32(1))

    @plsc.parallel_loop(0, NV, unroll=U)
    def _(i):
        v, bits = bits_at(i)
        m = bits >= t_cut
        outs[pl.ds(ooff + i * 16, 16)] = jnp.where(m, v, 0.0)


@functools.lru_cache(maxsize=None)
def _make_select(n_tok):
    ROWS_PER_W = n_tok // NW
    mesh = plsc.VectorSubcoreMesh(core_axis_name="c", subcore_axis_name="s",
                                  num_cores=2, num_subcores=16)

    @functools.partial(
        pl.kernel, mesh=mesh,
        out_type=jax.ShapeDtypeStruct((n_tok, D_SAE), jnp.float32),
        scratch_types=[
            pltpu.VMEM((2 * D_SAE,), jnp.float32),
            pltpu.VMEM((2 * D_SAE,), jnp.float32),
            pltpu.VMEM((HB,), jnp.int32),
            pltpu.SemaphoreType.DMA,
            pltpu.SemaphoreType.DMA,
            pltpu.SemaphoreType.DMA,
            pltpu.SemaphoreType.DMA,
        ],
        compiler_params=pltpu.CompilerParams(needs_layout_passes=False),
    )
    def select(pre_hbm, out_hbm, rows, outs, hist, si0, si1, so0, so1):
        wid = lax.axis_index("s") * 2 + lax.axis_index("c")
        base = wid * ROWS_PER_W
        isems = (si0, si1)
        osems = (so0, so1)

        pltpu.async_copy(pre_hbm.at[base], rows.at[pl.ds(0, D_SAE)], si0)

        def pair_body(r2, _):
            for b in range(2):
                r = 2 * r2 + b
                row = base + r
                roff = b * D_SAE
                pltpu.make_async_copy(
                    pre_hbm.at[row], rows.at[pl.ds(roff, D_SAE)],
                    isems[b]).wait()

                @pl.when(r + 1 < ROWS_PER_W)
                def _():
                    pltpu.async_copy(
                        pre_hbm.at[row + 1],
                        rows.at[pl.ds((1 - b) * D_SAE, D_SAE)], isems[1 - b])

                @pl.when(r >= 2)
                def _():
                    pltpu.make_async_copy(
                        outs.at[pl.ds(roff, D_SAE)], out_hbm.at[row - 2],
                        osems[b]).wait()

                _row_select(rows, outs, hist, roff, roff)
                pltpu.async_copy(outs.at[pl.ds(roff, D_SAE)],
                                 out_hbm.at[row], osems[b])
            return 0
        lax.fori_loop(0, ROWS_PER_W // 2, pair_body, 0)

        for b in range(2):
            row = base + ROWS_PER_W - 2 + b
            pltpu.make_async_copy(outs.at[pl.ds(b * D_SAE, D_SAE)],
                                  out_hbm.at[row], osems[b]).wait()

    return select


NCHUNK = 8


def kernel(x, W_enc, b_enc, W_dec, b_dec):
    cs = N_TOK // NCHUNK
    recs, enc_full = [], None
    for c in range(NCHUNK):
        pre_c = _encode(lax.slice_in_dim(x, c * cs, (c + 1) * cs), W_enc,
                        b_enc, b_dec)
        enc_c = _make_select(cs)(pre_c)
        rec_c, enc_full = _decode_assemble(enc_c, W_dec, b_dec, enc_full, c)
        recs.append(rec_c)
    return jnp.concatenate(recs, 0), enc_full


# speedup vs baseline: 1.4406x; 1.0121x over previous
import functools

import jax
import jax.numpy as jnp
from jax import lax
from jax.experimental import pallas as pl
from jax.experimental.pallas import tpu as pltpu
from jax.experimental.pallas import tpu_sc as plsc

D_MODEL = 768
D_SAE = 24576
K = 128
N_TOK = 8192

BR = 1024
BC = 512

_PREC = lax.Precision.DEFAULT


def _encode_body(x_ref, w_ref, benc_ref, bdec_ref, out_ref, bias_scr):
    i = pl.program_id(0)
    j = pl.program_id(1)

    @pl.when(i == 0)
    def _():
        bias_scr[0, pl.ds(j * BC, BC)] = (
            benc_ref[...]
            - lax.dot_general(bdec_ref[...], w_ref[...],
                              (((1,), (1,)), ((), ())), precision=_PREC)
        )[0]

    bias = bias_scr[0, pl.ds(j * BC, BC)]
    acts = lax.dot_general(x_ref[...], w_ref[...],
                           (((1,), (1,)), ((), ())), precision=_PREC)
    out_ref[...] = jnp.maximum(acts + bias[None, :], 0.0)


def _encode(x, W_enc, b_enc, b_dec):
    grid = (x.shape[0] // BR, D_SAE // BC)
    return pl.pallas_call(
        _encode_body,
        grid=grid,
        in_specs=[
            pl.BlockSpec((BR, D_MODEL), lambda i, j: (i, 0)),
            pl.BlockSpec((BC, D_MODEL), lambda i, j: (j, 0)),
            pl.BlockSpec((1, BC), lambda i, j: (0, j)),
            pl.BlockSpec((1, D_MODEL), lambda i, j: (0, 0)),
        ],
        out_specs=pl.BlockSpec((BR, BC), lambda i, j: (i, j)),
        out_shape=jax.ShapeDtypeStruct((x.shape[0], D_SAE), jnp.float32),
        scratch_shapes=[pltpu.VMEM((1, D_SAE), jnp.float32)],
    )(x, W_enc, b_enc.reshape(1, D_SAE), b_dec.reshape(1, D_MODEL))


def _decode_body(*refs):
    if len(refs) == 6:
        enc_ref, w_ref, bdec_ref, _prev, out_ref, encfull_ref = refs
    else:
        enc_ref, w_ref, bdec_ref, out_ref, encfull_ref = refs
    k = pl.program_id(1)
    encfull_ref[...] = enc_ref[...]
    acts = lax.dot_general(enc_ref[...], w_ref[...],
                           (((1,), (1,)), ((), ())), precision=_PREC)

    @pl.when(k == 0)
    def _():
        out_ref[...] = acts + bdec_ref[...]

    @pl.when(k != 0)
    def _():
        out_ref[...] += acts


def _decode_assemble(encoded_c, W_dec, b_dec, enc_prev, chunk):
    cs = encoded_c.shape[0]
    grid = (cs // BR, D_SAE // BC)
    row0 = chunk * cs // BR
    args = [encoded_c, W_dec, b_dec.reshape(1, D_MODEL)]
    aliases = {}
    if enc_prev is not None:
        args.append(enc_prev)
        aliases = {3: 1}
    return pl.pallas_call(
        _decode_body,
        grid=grid,
        in_specs=[
            pl.BlockSpec((BR, BC), lambda i, k: (i, k)),
            pl.BlockSpec((D_MODEL, BC), lambda i, k: (0, k)),
            pl.BlockSpec((1, D_MODEL), lambda i, k: (0, 0)),
        ] + ([pl.BlockSpec(memory_space=pl.MemorySpace.ANY)] if enc_prev is not None
             else []),
        out_specs=[
            pl.BlockSpec((BR, D_MODEL), lambda i, k: (i, 0)),
            pl.BlockSpec((BR, BC), lambda i, k: (row0 + i, k)),
        ],
        out_shape=[
            jax.ShapeDtypeStruct((cs, D_MODEL), jnp.float32),
            jax.ShapeDtypeStruct((N_TOK, D_SAE), jnp.float32),
        ],
        input_output_aliases=aliases,
    )(*args)


NW = 32
NV = D_SAE // 16
U = 8
HB = 8192


def _scan_top(hist, start_block, target):
    iota16 = lax.iota(jnp.int32, 16)

    def sc_cond(c):
        i, cum = c
        return jnp.logical_and(cum < target, i >= 0)

    def sc_body(c):
        i, cum = c
        return i - 1, cum + jnp.sum(hist[pl.ds(i * 16, 16)])

    i_end, cum_end = lax.while_loop(
        sc_cond, sc_body, (start_block, jnp.int32(0)))
    found = cum_end >= target
    iv = jnp.maximum(i_end + 1, 0)
    h = hist[pl.ds(iv * 16, 16)]
    cum_above_blk = cum_end - jnp.sum(h)
    suffix = lax.rev(jnp.cumsum(lax.rev(h, (0,))), (0,)) + cum_above_blk
    m = suffix >= target
    lane = jnp.sum(m.astype(jnp.int32)) - 1
    bucket = iv * 16 + lane
    sfx_lane = cum_above_blk + jnp.sum(jnp.where(iota16 >= lane, h, 0))
    h_lane = jnp.sum(jnp.where(iota16 == lane, h, 0))
    above = sfx_lane - h_lane
    return found, bucket, above, h_lane


def _zero_hist(hist, nbuckets):
    @plsc.parallel_loop(0, nbuckets // 16, unroll=U)
    def _(i):
        hist[pl.ds(i * 16, 16)] = jnp.zeros((16,), jnp.int32)


def _row_select(rows, outs, hist, roff, ooff):
    ones16 = jnp.ones((16,), jnp.int32)

    def bits_at(i):
        v = rows[pl.ds(roff + i * 16, 16)]
        return v, lax.bitcast_convert_type(v, jnp.int32)

    _zero_hist(hist, 8192)

    @plsc.parallel_loop(0, NV, unroll=U, carry=jnp.zeros((16,), jnp.int32))
    def mbv(i, mb):
        _, bits = bits_at(i)
        b = lax.shift_right_logical(bits, 18)
        plsc.addupdate_scatter(hist, [b], ones16, mask=bits >= 1)
        return jnp.maximum(mb, jnp.where(bits >= 1, b, 0))

    max_blk = lax.shift_right_logical(jnp.max(mbv), 4)
    found1, b1, above1, cnt1 = _scan_top(hist, max_blk, jnp.int32(K))
    need2 = jnp.int32(K) - above1

    def _refine_l2(_):
        _zero_hist(hist, 512)

        @plsc.parallel_loop(0, NV, unroll=U)
        def _(i):
            _, bits = bits_at(i)
            m = lax.shift_right_logical(bits, 18) == b1
            b = jnp.bitwise_and(lax.shift_right_logical(bits, 9),
                                jnp.int32(0x1FF))
            plsc.addupdate_scatter(hist, [b], ones16, mask=m)

        _f2, b2, above2, cnt2 = _scan_top(hist, jnp.int32(31), need2)
        prefix22 = jnp.bitwise_or(lax.shift_left(b1, 9), b2)
        need3 = need2 - above2

        def _refine_l3(_):
            _zero_hist(hist, 512)

            @plsc.parallel_loop(0, NV, unroll=U)
            def _(i):
                _, bits = bits_at(i)
                m = lax.shift_right_logical(bits, 9) == prefix22
                b = jnp.bitwise_and(bits, jnp.int32(0x1FF))
                plsc.addupdate_scatter(hist, [b], ones16, mask=m)

            _f3, b3, _a3, _c3 = _scan_top(hist, jnp.int32(31), need3)
            return jnp.bitwise_or(lax.shift_left(prefix22, 9), b3)

        return lax.cond(cnt2 == need3, lambda _: lax.shift_left(prefix22, 9),
                        _refine_l3, 0)

    skip_l2 = jnp.logical_or(jnp.logical_not(found1), cnt1 == need2)
    t_cut = lax.cond(skip_l2, lambda _: lax.shift_left(b1, 18), _refine_l2, 0)
    t_cut = jnp.where(found1, jnp.maximum(t_cut, 1), jnp.int32(1))

    @plsc.parallel_loop(0, NV, unroll=U)
    def _(i):
        v, bits = bits_at(i)
        m = bits >= t_cut
        outs[pl.ds(ooff + i * 16, 16)] = jnp.where(m, v, 0.0)


@functools.lru_cache(maxsize=None)
def _make_select(n_tok):
    ROWS_PER_W = n_tok // NW
    mesh = plsc.VectorSubcoreMesh(core_axis_name="c", subcore_axis_name="s",
                                  num_cores=2, num_subcores=16)

    @functools.partial(
        pl.kernel, mesh=mesh,
        out_type=jax.ShapeDtypeStruct((n_tok, D_SAE), jnp.float32),
        scratch_types=[
            pltpu.VMEM((2 * D_SAE,), jnp.float32),
            pltpu.VMEM((2 * D_SAE,), jnp.float32),
            pltpu.VMEM((HB,), jnp.int32),
            pltpu.SemaphoreType.DMA,
            pltpu.SemaphoreType.DMA,
            pltpu.SemaphoreType.DMA,
            pltpu.SemaphoreType.DMA,
        ],
        compiler_params=pltpu.CompilerParams(needs_layout_passes=False),
    )
    def select(pre_hbm, out_hbm, rows, outs, hist, si0, si1, so0, so1):
        wid = lax.axis_index("s") * 2 + lax.axis_index("c")
        base = wid * ROWS_PER_W
        isems = (si0, si1)
        osems = (so0, so1)

        pltpu.async_copy(pre_hbm.at[base], rows.at[pl.ds(0, D_SAE)], si0)

        def pair_body(r2, _):
            for b in range(2):
                r = 2 * r2 + b
                row = base + r
                roff = b * D_SAE
                pltpu.make_async_copy(
                    pre_hbm.at[row], rows.at[pl.ds(roff, D_SAE)],
                    isems[b]).wait()

                @pl.when(r + 1 < ROWS_PER_W)
                def _():
                    pltpu.async_copy(
                        pre_hbm.at[row + 1],
                        rows.at[pl.ds((1 - b) * D_SAE, D_SAE)], isems[1 - b])

                @pl.when(r >= 2)
                def _():
                    pltpu.make_async_copy(
                        outs.at[pl.ds(roff, D_SAE)], out_hbm.at[row - 2],
                        osems[b]).wait()

                _row_select(rows, outs, hist, roff, roff)
                pltpu.async_copy(outs.at[pl.ds(roff, D_SAE)],
                                 out_hbm.at[row], osems[b])
            return 0
        lax.fori_loop(0, ROWS_PER_W // 2, pair_body, 0)

        for b in range(2):
            row = base + ROWS_PER_W - 2 + b
            pltpu.make_async_copy(outs.at[pl.ds(b * D_SAE, D_SAE)],
                                  out_hbm.at[row], osems[b]).wait()

    return select


NCHUNK = 4


def kernel(x, W_enc, b_enc, W_dec, b_dec):
    cs = N_TOK // NCHUNK
    recs, enc_full = [], None
    for c in range(NCHUNK):
        pre_c = _encode(lax.slice_in_dim(x, c * cs, (c + 1) * cs), W_enc,
                        b_enc, b_dec)
        enc_c = _make_select(cs)(pre_c)
        rec_c, enc_full = _decode_assemble(enc_c, W_dec, b_dec, enc_full, c)
        recs.append(rec_c)
    return jnp.concatenate(recs, 0), enc_full


# speedup vs baseline: 1.8185x; 1.2623x over previous
import functools

import jax
import jax.numpy as jnp
from jax import lax
from jax.experimental import pallas as pl
from jax.experimental.pallas import tpu as pltpu
from jax.experimental.pallas import tpu_sc as plsc

D_MODEL = 768
D_SAE = 24576
K = 128
N_TOK = 8192

BR = 1024
BC = 512

_PREC = lax.Precision.DEFAULT


def _encode_body(x_ref, w_ref, benc_ref, bdec_ref, out_ref, bias_scr):
    i = pl.program_id(0)
    j = pl.program_id(1)

    @pl.when(i == 0)
    def _():
        bias_scr[0, pl.ds(j * BC, BC)] = (
            benc_ref[...]
            - lax.dot_general(bdec_ref[...], w_ref[...],
                              (((1,), (1,)), ((), ())), precision=_PREC)
        )[0]

    bias = bias_scr[0, pl.ds(j * BC, BC)]
    acts = lax.dot_general(x_ref[...], w_ref[...],
                           (((1,), (1,)), ((), ())), precision=_PREC)
    out_ref[...] = jnp.maximum(acts + bias[None, :], 0.0)


def _encode(x, W_enc, b_enc, b_dec):
    grid = (x.shape[0] // BR, D_SAE // BC)
    return pl.pallas_call(
        _encode_body,
        grid=grid,
        in_specs=[
            pl.BlockSpec((BR, D_MODEL), lambda i, j: (i, 0)),
            pl.BlockSpec((BC, D_MODEL), lambda i, j: (j, 0)),
            pl.BlockSpec((1, BC), lambda i, j: (0, j)),
            pl.BlockSpec((1, D_MODEL), lambda i, j: (0, 0)),
        ],
        out_specs=pl.BlockSpec((BR, BC), lambda i, j: (i, j)),
        out_shape=jax.ShapeDtypeStruct((x.shape[0], D_SAE), jnp.float32),
        scratch_shapes=[pltpu.VMEM((1, D_SAE), jnp.float32)],
    )(x, W_enc, b_enc.reshape(1, D_SAE), b_dec.reshape(1, D_MODEL))


def _decode_body(*refs):
    if len(refs) == 6:
        enc_ref, w_ref, bdec_ref, _prev, out_ref, encfull_ref = refs
    else:
        enc_ref, w_ref, bdec_ref, out_ref, encfull_ref = refs
    k = pl.program_id(1)
    encfull_ref[...] = enc_ref[...]
    acts = lax.dot_general(enc_ref[...], w_ref[...],
                           (((1,), (1,)), ((), ())), precision=_PREC)

    @pl.when(k == 0)
    def _():
        out_ref[...] = acts + bdec_ref[...]

    @pl.when(k != 0)
    def _():
        out_ref[...] += acts


def _decode_assemble(encoded_c, W_dec, b_dec, enc_prev, chunk):
    cs = encoded_c.shape[0]
    grid = (cs // BR, D_SAE // BC)
    row0 = chunk * cs // BR
    args = [encoded_c, W_dec, b_dec.reshape(1, D_MODEL)]
    aliases = {}
    if enc_prev is not None:
        args.append(enc_prev)
        aliases = {3: 1}
    return pl.pallas_call(
        _decode_body,
        grid=grid,
        in_specs=[
            pl.BlockSpec((BR, BC), lambda i, k: (i, k)),
            pl.BlockSpec((D_MODEL, BC), lambda i, k: (0, k)),
            pl.BlockSpec((1, D_MODEL), lambda i, k: (0, 0)),
        ] + ([pl.BlockSpec(memory_space=pl.MemorySpace.ANY)] if enc_prev is not None
             else []),
        out_specs=[
            pl.BlockSpec((BR, D_MODEL), lambda i, k: (i, 0)),
            pl.BlockSpec((BR, BC), lambda i, k: (row0 + i, k)),
        ],
        out_shape=[
            jax.ShapeDtypeStruct((cs, D_MODEL), jnp.float32),
            jax.ShapeDtypeStruct((N_TOK, D_SAE), jnp.float32),
        ],
        input_output_aliases=aliases,
    )(*args)


NW = 32
NV = D_SAE // 16
U = 8
HB = 8192


def _scan_top(hist, start_block, target):
    iota16 = lax.iota(jnp.int32, 16)

    def sc_cond(c):
        i, cum = c
        return jnp.logical_and(cum < target, i >= 0)

    def sc_body(c):
        i, cum = c
        return i - 1, cum + jnp.sum(hist[pl.ds(i * 16, 16)])

    i_end, cum_end = lax.while_loop(
        sc_cond, sc_body, (start_block, jnp.int32(0)))
    found = cum_end >= target
    iv = jnp.maximum(i_end + 1, 0)
    h = hist[pl.ds(iv * 16, 16)]
    cum_above_blk = cum_end - jnp.sum(h)
    suffix = lax.rev(jnp.cumsum(lax.rev(h, (0,))), (0,)) + cum_above_blk
    m = suffix >= target
    lane = jnp.sum(m.astype(jnp.int32)) - 1
    bucket = iv * 16 + lane
    sfx_lane = cum_above_blk + jnp.sum(jnp.where(iota16 >= lane, h, 0))
    h_lane = jnp.sum(jnp.where(iota16 == lane, h, 0))
    above = sfx_lane - h_lane
    return found, bucket, above, h_lane


def _zero_hist(hist, nbuckets):
    @plsc.parallel_loop(0, nbuckets // 16, unroll=U)
    def _(i):
        hist[pl.ds(i * 16, 16)] = jnp.zeros((16,), jnp.int32)


def _row_select(rows, outs, hist, roff, ooff):
    ones16 = jnp.ones((16,), jnp.int32)

    def bits_at(i):
        v = rows[pl.ds(roff + i * 16, 16)]
        return v, lax.bitcast_convert_type(v, jnp.int32)

    _zero_hist(hist, 8192)

    @plsc.parallel_loop(0, NV, unroll=U, carry=jnp.zeros((16,), jnp.int32))
    def mbv(i, mb):
        _, bits = bits_at(i)
        b = lax.shift_right_logical(bits, 18)
        plsc.addupdate_scatter(hist, [b], ones16, mask=bits >= 1)
        return jnp.maximum(mb, jnp.where(bits >= 1, b, 0))

    max_blk = lax.shift_right_logical(jnp.max(mbv), 4)
    found1, b1, above1, cnt1 = _scan_top(hist, max_blk, jnp.int32(K))
    need2 = jnp.int32(K) - above1

    def _refine_l2(_):
        _zero_hist(hist, 512)

        @plsc.parallel_loop(0, NV, unroll=U)
        def _(i):
            _, bits = bits_at(i)
            m = lax.shift_right_logical(bits, 18) == b1
            b = jnp.bitwise_and(lax.shift_right_logical(bits, 9),
                                jnp.int32(0x1FF))
            plsc.addupdate_scatter(hist, [b], ones16, mask=m)

        _f2, b2, above2, cnt2 = _scan_top(hist, jnp.int32(31), need2)
        prefix22 = jnp.bitwise_or(lax.shift_left(b1, 9), b2)
        need3 = need2 - above2

        def _refine_l3(_):
            _zero_hist(hist, 512)

            @plsc.parallel_loop(0, NV, unroll=U)
            def _(i):
                _, bits = bits_at(i)
                m = lax.shift_right_logical(bits, 9) == prefix22
                b = jnp.bitwise_and(bits, jnp.int32(0x1FF))
                plsc.addupdate_scatter(hist, [b], ones16, mask=m)

            _f3, b3, _a3, _c3 = _scan_top(hist, jnp.int32(31), need3)
            return jnp.bitwise_or(lax.shift_left(prefix22, 9), b3)

        return lax.cond(cnt2 == need3, lambda _: lax.shift_left(prefix22, 9),
                        _refine_l3, 0)

    skip_l2 = jnp.logical_or(jnp.logical_not(found1), cnt1 == need2)
    t_cut = lax.cond(skip_l2, lambda _: lax.shift_left(b1, 18), _refine_l2, 0)
    t_cut = jnp.where(found1, jnp.maximum(t_cut, 1), jnp.int32(1))

    @plsc.parallel_loop(0, NV, unroll=U)
    def _(i):
        v, bits = bits_at(i)
        m = bits >= t_cut
        outs[pl.ds(ooff + i * 16, 16)] = jnp.where(m, v, 0.0)


@functools.lru_cache(maxsize=None)
def _make_select(n_tok):
    ROWS_PER_W = n_tok // NW
    mesh = plsc.VectorSubcoreMesh(core_axis_name="c", subcore_axis_name="s",
                                  num_cores=2, num_subcores=16)

    @functools.partial(
        pl.kernel, mesh=mesh,
        out_type=jax.ShapeDtypeStruct((n_tok, D_SAE), jnp.float32),
        scratch_types=[
            pltpu.VMEM((2 * D_SAE,), jnp.float32),
            pltpu.VMEM((2 * D_SAE,), jnp.float32),
            pltpu.VMEM((HB,), jnp.int32),
            pltpu.SemaphoreType.DMA,
            pltpu.SemaphoreType.DMA,
            pltpu.SemaphoreType.DMA,
            pltpu.SemaphoreType.DMA,
        ],
        compiler_params=pltpu.CompilerParams(needs_layout_passes=False),
    )
    def select(pre_hbm, out_hbm, rows, outs, hist, si0, si1, so0, so1):
        wid = lax.axis_index("s") * 2 + lax.axis_index("c")
        base = wid * ROWS_PER_W
        isems = (si0, si1)
        osems = (so0, so1)

        pltpu.async_copy(pre_hbm.at[base], rows.at[pl.ds(0, D_SAE)], si0)

        def pair_body(r2, _):
            for b in range(2):
                r = 2 * r2 + b
                row = base + r
                roff = b * D_SAE
                pltpu.make_async_copy(
                    pre_hbm.at[row], rows.at[pl.ds(roff, D_SAE)],
                    isems[b]).wait()

                @pl.when(r + 1 < ROWS_PER_W)
                def _():
                    pltpu.async_copy(
                        pre_hbm.at[row + 1],
                        rows.at[pl.ds((1 - b) * D_SAE, D_SAE)], isems[1 - b])

                @pl.when(r >= 2)
                def _():
                    pltpu.make_async_copy(
                        outs.at[pl.ds(roff, D_SAE)], out_hbm.at[row - 2],
                        osems[b]).wait()

                _row_select(rows, outs, hist, roff, roff)
                pltpu.async_copy(outs.at[pl.ds(roff, D_SAE)],
                                 out_hbm.at[row], osems[b])
            return 0
        lax.fori_loop(0, ROWS_PER_W // 2, pair_body, 0)

        for b in range(2):
            row = base + ROWS_PER_W - 2 + b
            pltpu.make_async_copy(outs.at[pl.ds(b * D_SAE, D_SAE)],
                                  out_hbm.at[row], osems[b]).wait()

    return select


NCHUNK = 16


def kernel(x, W_enc, b_enc, W_dec, b_dec):
    cs = N_TOK // NCHUNK
    recs, enc_full = [], None
    for c in range(NCHUNK):
        pre_c = _encode(lax.slice_in_dim(x, c * cs, (c + 1) * cs), W_enc,
                        b_enc, b_dec)
        enc_c = _make_select(cs)(pre_c)
        rec_c, enc_full = _decode_assemble(enc_c, W_dec, b_dec, enc_full, c)
        recs.append(rec_c)
    return jnp.concatenate(recs, 0), enc_full
